# Initial kernel scaffold; baseline (speedup 1.0000x reference)
import jax
import jax.numpy as jnp
from jax.experimental import pallas as pl


def kernel(pos, edge_attr, edge_index, enc_W, enc_b, dec_W, dec_b, e_W1, e_b1, e_W2, e_b2, n_W1, n_b1, n_W2, n_b2):
    raise NotImplementedError("write your pallas kernel here")


# speedup vs baseline: 2.5976x; 2.5976x over previous
import functools

import jax
import jax.numpy as jnp
from jax import lax
from jax.experimental import pallas as pl
from jax.experimental.pallas import tpu as pltpu
from jax.experimental.pallas import tpu_sc as plsc

H = 128
N = 10000
E = 320000
NLAYERS = 6

NC = 2
NS = 16
NW = NC * NS
E_PER_W = E // NW
CHUNK = 80
N_CHUNKS = E_PER_W // CHUNK
ROWS_PER_SUB = 624
TAIL_ROWS = N - NS * ROWS_PER_SUB
CW = 128

BE = 1280
BN = 2000

_mesh = functools.partial(
    plsc.VectorSubcoreMesh, core_axis_name="c", subcore_axis_name="s"
)


def _gather_sum(x1, x2, dst, src):
  @functools.partial(
      pl.kernel,
      out_type=jax.ShapeDtypeStruct((E, H), jnp.float32),
      mesh=_mesh(),
      scratch_types=[
          pltpu.VMEM((CHUNK,), jnp.int32),
          pltpu.VMEM((CHUNK,), jnp.int32),
          pltpu.VMEM((CHUNK, H), jnp.float32),
          pltpu.VMEM((CHUNK, H), jnp.float32),
          pltpu.SemaphoreType.DMA,
      ],
  )
  def k(x1_hbm, x2_hbm, dst_hbm, src_hbm, out_hbm, idx_d, idx_s, rows_a,
        rows_b, sem):
    w = lax.axis_index("s") * NC + lax.axis_index("c")
    base = w * E_PER_W

    def step(kk, carry):
      off = base + kk * CHUNK
      pltpu.sync_copy(dst_hbm.at[pl.ds(off, CHUNK)], idx_d)
      pltpu.sync_copy(src_hbm.at[pl.ds(off, CHUNK)], idx_s)
      cp1 = pltpu.async_copy(x1_hbm.at[idx_d], rows_a, sem)
      cp2 = pltpu.async_copy(x2_hbm.at[idx_s], rows_b, sem)
      cp1.wait()
      cp2.wait()

      def add_row(r, c2):
        for cc in range(H // 16):
          sl = pl.ds(cc * 16, 16)
          plsc.addupdate(rows_a.at[r, sl], rows_b[r, sl])
        return c2

      lax.fori_loop(0, CHUNK, add_row, 0)
      pltpu.sync_copy(rows_a, out_hbm.at[pl.ds(off, CHUNK)])
      return carry

    lax.fori_loop(0, N_CHUNKS, step, 0)

  return k(x1, x2, dst, src)


def _scatter_add(m, dst, zrows):
  @functools.partial(
      pl.kernel,
      out_type=jax.ShapeDtypeStruct((NC, N, H), jnp.float32),
      mesh=_mesh(),
      scratch_types=[
          pltpu.VMEM((CHUNK,), jnp.int32),
          pltpu.VMEM((CHUNK, H), jnp.float32),
          pltpu.VMEM_SHARED((N, H), jnp.float32),
      ],
  )
  def k(m_hbm, dst_hbm, z_hbm, out_hbm, idx_v, vals, acc):
    c = lax.axis_index("c")
    s = lax.axis_index("s")
    rsl = pl.ds(s * ROWS_PER_SUB, ROWS_PER_SUB)
    tsl = pl.ds(NS * ROWS_PER_SUB, TAIL_ROWS)
    pltpu.sync_copy(z_hbm, acc.at[rsl])
    @pl.when(s == 0)
    def _():
      pltpu.sync_copy(z_hbm.at[pl.ds(0, TAIL_ROWS)], acc.at[tsl])
    plsc.subcore_barrier()
    base = c * (E // NC) + s * E_PER_W

    def step(kk, carry):
      off = base + kk * CHUNK
      pltpu.sync_copy(dst_hbm.at[pl.ds(off, CHUNK)], idx_v)
      pltpu.sync_copy(m_hbm.at[pl.ds(off, CHUNK)], vals)
      pltpu.sync_copy(vals, acc.at[idx_v], add=True)
      return carry

    lax.fori_loop(0, N_CHUNKS, step, 0)
    plsc.subcore_barrier()
    pltpu.sync_copy(acc.at[rsl], out_hbm.at[c].at[rsl])
    @pl.when(s == 0)
    def _():
      pltpu.sync_copy(acc.at[tsl], out_hbm.at[c].at[tsl])

  return k(m, dst, zrows)


def _counts(dst, zrows):
  @functools.partial(
      pl.kernel,
      out_type=jax.ShapeDtypeStruct((NC, N, CW), jnp.float32),
      mesh=_mesh(),
      scratch_types=[
          pltpu.VMEM((CHUNK,), jnp.int32),
          pltpu.VMEM((CHUNK, CW), jnp.float32),
          pltpu.VMEM_SHARED((N, CW), jnp.float32),
      ],
  )
  def k(dst_hbm, z_hbm, out_hbm, idx_v, ones_v, acc):
    c = lax.axis_index("c")
    s = lax.axis_index("s")
    rsl = pl.ds(s * ROWS_PER_SUB, ROWS_PER_SUB)
    tsl = pl.ds(NS * ROWS_PER_SUB, TAIL_ROWS)
    pltpu.sync_copy(z_hbm, acc.at[rsl])
    @pl.when(s == 0)
    def _():
      pltpu.sync_copy(z_hbm.at[pl.ds(0, TAIL_ROWS)], acc.at[tsl])
    one = jnp.ones((16,), jnp.float32)

    def fill_row(r, carry):
      for cc in range(CW // 16):
        ones_v[r, pl.ds(cc * 16, 16)] = one
      return carry

    lax.fori_loop(0, CHUNK, fill_row, 0)
    plsc.subcore_barrier()
    base = c * (E // NC) + s * E_PER_W

    def step(kk, carry):
      off = base + kk * CHUNK
      pltpu.sync_copy(dst_hbm.at[pl.ds(off, CHUNK)], idx_v)
      pltpu.sync_copy(ones_v, acc.at[idx_v], add=True)
      return carry

    lax.fori_loop(0, N_CHUNKS, step, 0)
    plsc.subcore_barrier()
    pltpu.sync_copy(acc.at[rsl], out_hbm.at[c].at[rsl])
    @pl.when(s == 0)
    def _():
      pltpu.sync_copy(acc.at[tsl], out_hbm.at[c].at[tsl])

  return k(dst, zrows)


def _edge_mlp(g, ef, w1c, b1, w2, b2):
  def body(g_ref, ef_ref, w1_ref, b1_ref, w2_ref, b2_ref, m_ref, efo_ref):
    efv = ef_ref[...]
    pre = g_ref[...] + jnp.dot(
        efv, w1_ref[...], preferred_element_type=jnp.float32) + b1_ref[...]
    h = jnp.maximum(pre, 0.0)
    m = jnp.dot(h, w2_ref[...],
                preferred_element_type=jnp.float32) + b2_ref[...]
    m_ref[...] = m
    efo_ref[...] = efv + m

  return pl.pallas_call(
      body,
      grid=(E // BE,),
      in_specs=[
          pl.BlockSpec((BE, H), lambda i: (i, 0)),
          pl.BlockSpec((BE, H), lambda i: (i, 0)),
          pl.BlockSpec((H, H), lambda i: (0, 0)),
          pl.BlockSpec((1, H), lambda i: (0, 0)),
          pl.BlockSpec((H, H), lambda i: (0, 0)),
          pl.BlockSpec((1, H), lambda i: (0, 0)),
      ],
      out_specs=[pl.BlockSpec((BE, H), lambda i: (i, 0))] * 2,
      out_shape=[jax.ShapeDtypeStruct((E, H), jnp.float32)] * 2,
  )(g, ef, w1c, b1, w2, b2)


def _edge_mlp0(ea_t, enc_w, enc_b, w1c, b1, w2, b2):
  def body(ea_ref, ew_ref, eb_ref, w1_ref, b1_ref, w2_ref, b2_ref, m_ref,
           efo_ref):
    ef = lax.dot_general(
        ea_ref[...], ew_ref[...], (((0,), (0,)), ((), ())),
        preferred_element_type=jnp.float32) + eb_ref[...]
    pre = jnp.dot(ef, w1_ref[...],
                  preferred_element_type=jnp.float32) + b1_ref[...]
    h = jnp.maximum(pre, 0.0)
    m = jnp.dot(h, w2_ref[...],
                preferred_element_type=jnp.float32) + b2_ref[...]
    m_ref[...] = m
    efo_ref[...] = ef + m

  return pl.pallas_call(
      body,
      grid=(E // BE,),
      in_specs=[
          pl.BlockSpec((3, BE), lambda i: (0, i)),
          pl.BlockSpec((3, H), lambda i: (0, 0)),
          pl.BlockSpec((1, H), lambda i: (0, 0)),
          pl.BlockSpec((H, H), lambda i: (0, 0)),
          pl.BlockSpec((1, H), lambda i: (0, 0)),
          pl.BlockSpec((H, H), lambda i: (0, 0)),
          pl.BlockSpec((1, H), lambda i: (0, 0)),
      ],
      out_specs=[pl.BlockSpec((BE, H), lambda i: (i, 0))] * 2,
      out_shape=[jax.ShapeDtypeStruct((E, H), jnp.float32)] * 2,
  )(ea_t, enc_w, enc_b, w1c, b1, w2, b2)


def _node_mlp(x, p, cnts, wa, wb, b1, w2, b2, w1a_n, w1b_n):
  with_next = w1a_n is not None

  def body(x_ref, p_ref, c_ref, wa_ref, wb_ref, b1_ref, w2_ref, b2_ref,
           *rest):
    cnt = c_ref[0, :, 0:1] + c_ref[1, :, 0:1]
    inv = 1.0 / jnp.maximum(cnt, 1.0)
    aggr = (p_ref[0] + p_ref[1]) * inv
    xv = x_ref[...]
    h = jnp.maximum(
        jnp.dot(xv, wa_ref[...], preferred_element_type=jnp.float32)
        + jnp.dot(aggr, wb_ref[...], preferred_element_type=jnp.float32)
        + b1_ref[...], 0.0)
    xo = xv + jnp.dot(h, w2_ref[...],
                      preferred_element_type=jnp.float32) + b2_ref[...]
    if with_next:
      w1a_ref, w1b_ref, xo_ref, x1_ref, x2_ref = rest
      xo_ref[...] = xo
      x1_ref[...] = jnp.dot(xo, w1a_ref[...],
                            preferred_element_type=jnp.float32)
      x2_ref[...] = jnp.dot(xo, w1b_ref[...],
                            preferred_element_type=jnp.float32)
    else:
      rest[0][...] = xo

  in_specs = [
      pl.BlockSpec((BN, H), lambda i: (i, 0)),
      pl.BlockSpec((NC, BN, H), lambda i: (0, i, 0)),
      pl.BlockSpec((NC, BN, CW), lambda i: (0, i, 0)),
      pl.BlockSpec((H, H), lambda i: (0, 0)),
      pl.BlockSpec((H, H), lambda i: (0, 0)),
      pl.BlockSpec((1, H), lambda i: (0, 0)),
      pl.BlockSpec((H, H), lambda i: (0, 0)),
      pl.BlockSpec((1, H), lambda i: (0, 0)),
  ]
  args = [x, p, cnts, wa, wb, b1, w2, b2]
  n_out = 1
  if with_next:
    in_specs += [pl.BlockSpec((H, H), lambda i: (0, 0))] * 2
    args += [w1a_n, w1b_n]
    n_out = 3
  out = pl.pallas_call(
      body,
      grid=(N // BN,),
      in_specs=in_specs,
      out_specs=[pl.BlockSpec((BN, H), lambda i: (i, 0))] * n_out,
      out_shape=[jax.ShapeDtypeStruct((N, H), jnp.float32)] * n_out,
  )(*args)
  return out


def _decode(x, w_pad, b_pad):
  def body(x_ref, w_ref, b_ref, o_ref):
    out = jnp.dot(x_ref[...], w_ref[...],
                  preferred_element_type=jnp.float32) + b_ref[...]
    ss = jnp.sum(out * out, axis=1, keepdims=True)
    o_ref[...] = out / jnp.maximum(jnp.sqrt(ss), 1e-12)

  return pl.pallas_call(
      body,
      grid=(N // BN,),
      in_specs=[
          pl.BlockSpec((BN, H), lambda i: (i, 0)),
          pl.BlockSpec((H, H), lambda i: (0, 0)),
          pl.BlockSpec((1, H), lambda i: (0, 0)),
      ],
      out_specs=pl.BlockSpec((BN, H), lambda i: (i, 0)),
      out_shape=jax.ShapeDtypeStruct((N, H), jnp.float32),
  )(x, w_pad, b_pad)


def kernel(pos, edge_attr, edge_index, enc_W, enc_b, dec_W, dec_b, e_W1,
           e_b1, e_W2, e_b2, n_W1, n_b1, n_W2, n_b2):
  del pos
  f32 = jnp.float32
  src = edge_index[0]
  dst = edge_index[1]
  ea_t = edge_attr.T
  enc_b2 = enc_b.reshape(1, H)
  zrows = jnp.zeros((ROWS_PER_SUB, H), f32)
  dec_w_pad = jnp.zeros((H, H), f32).at[:, :3].set(dec_W)
  dec_b_pad = jnp.zeros((1, H), f32).at[0, :3].set(dec_b)

  cnts = _counts(dst, zrows)

  x = jnp.zeros((N, H), f32)
  ef = None
  g = None
  for i in range(NLAYERS):
    w1c = e_W1[i, 2 * H:3 * H]
    b1 = e_b1[i].reshape(1, H)
    w2 = e_W2[i]
    b2 = e_b2[i].reshape(1, H)
    if i == 0:
      m, ef = _edge_mlp0(ea_t, enc_W, enc_b2, w1c, b1, w2, b2)
    else:
      m, ef = _edge_mlp(g, ef, w1c, b1, w2, b2)
    p = _scatter_add(m, dst, zrows)
    wa = n_W1[i, :H]
    wb = n_W1[i, H:]
    nb1 = n_b1[i].reshape(1, H)
    nw2 = n_W2[i]
    nb2 = n_b2[i].reshape(1, H)
    if i < NLAYERS - 1:
      x, x1t, x2t = _node_mlp(x, p, cnts, wa, wb, nb1, nw2, nb2,
                              e_W1[i + 1, :H], e_W1[i + 1, H:2 * H])
      g = _gather_sum(x1t, x2t, dst, src)
    else:
      (x,) = _node_mlp(x, p, cnts, wa, wb, nb1, nw2, nb2, None, None)
  out = _decode(x, dec_w_pad, dec_b_pad)
  return out[:, :3]


# speedup vs baseline: 4.1360x; 1.5922x over previous
import functools

import jax
import jax.numpy as jnp
from jax import lax
from jax.experimental import pallas as pl
from jax.experimental.pallas import tpu as pltpu
from jax.experimental.pallas import tpu_sc as plsc

H = 128
N = 10000
E = 320000
NLAYERS = 6

NC = 2
NS = 16
NW = NC * NS
E_PER_W = E // NW
CHUNK = 80
N_CHUNKS = E_PER_W // CHUNK
BL = 80
N_BLOCKS = E_PER_W // BL
ROWS_PER_SUB = 624
TAIL_ROWS = N - NS * ROWS_PER_SUB
CW = 128

BE = 1280
BN = 2000

_mesh = functools.partial(
    plsc.VectorSubcoreMesh, core_axis_name="c", subcore_axis_name="s"
)


def _gather_sum(x1, x2, dst, src):
  @functools.partial(
      pl.kernel,
      out_type=jax.ShapeDtypeStruct((E, H), jnp.float32),
      mesh=_mesh(),
      scratch_types=[
          pltpu.VMEM((E_PER_W,), jnp.int32),
          pltpu.VMEM((E_PER_W,), jnp.int32),
          pltpu.VMEM((CHUNK, H), jnp.float32),
          pltpu.VMEM((CHUNK, H), jnp.float32),
          pltpu.VMEM((CHUNK, H), jnp.float32),
          pltpu.VMEM((CHUNK, H), jnp.float32),
          pltpu.SemaphoreType.DMA,
          pltpu.SemaphoreType.DMA,
          pltpu.SemaphoreType.DMA,
          pltpu.SemaphoreType.DMA,
      ],
  )
  def k(x1_hbm, x2_hbm, dst_hbm, src_hbm, out_hbm, idx_d, idx_s, ra0, rb0,
        ra1, rb1, g0, g1, s0, s1):
    w = lax.axis_index("s") * NC + lax.axis_index("c")
    base = w * E_PER_W
    pltpu.sync_copy(dst_hbm.at[pl.ds(base, E_PER_W)], idx_d)
    pltpu.sync_copy(src_hbm.at[pl.ds(base, E_PER_W)], idx_s)
    ra = (ra0, ra1)
    rb = (rb0, rb1)
    gs = (g0, g1)
    ss = (s0, s1)

    def fire(kk, b):
      isl = pl.ds(kk * CHUNK, CHUNK)
      pltpu.async_copy(x1_hbm.at[idx_d.at[isl]], ra[b], gs[b])
      pltpu.async_copy(x2_hbm.at[idx_s.at[isl]], rb[b], gs[b])

    def drain_add(kk, b):
      pltpu.make_async_copy(x1_hbm.at[idx_d.at[pl.ds(0, CHUNK)]], ra[b],
                            gs[b]).wait()
      pltpu.make_async_copy(x2_hbm.at[idx_s.at[pl.ds(0, CHUNK)]], rb[b],
                            gs[b]).wait()

      def add_row(r, c2):
        for cc in range(H // 16):
          sl = pl.ds(cc * 16, 16)
          plsc.addupdate(ra[b].at[r, sl], rb[b][r, sl])
        return c2

      lax.fori_loop(0, CHUNK, add_row, 0)
      pltpu.async_copy(ra[b], out_hbm.at[pl.ds(base + kk * CHUNK, CHUNK)],
                       ss[b])

    def wait_store(b):
      pltpu.make_async_copy(ra[b], out_hbm.at[pl.ds(base, CHUNK)],
                            ss[b]).wait()

    fire(0, 0)

    def pair(i, carry):
      k0 = 2 * i

      @pl.when(i > 0)
      def _():
        wait_store(1)

      fire(k0 + 1, 1)
      drain_add(k0, 0)

      @pl.when(k0 + 2 < N_CHUNKS)
      def _():
        wait_store(0)
        fire(k0 + 2, 0)

      drain_add(k0 + 1, 1)
      return carry

    lax.fori_loop(0, N_CHUNKS // 2, pair, 0)
    drain_add(N_CHUNKS - 1, 0)
    wait_store(0)
    wait_store(1)

  return k(x1, x2, dst, src)


def _scatter_add(m, dst3d, zrows):
  SUB = BL // CHUNK

  @functools.partial(
      pl.kernel,
      out_type=jax.ShapeDtypeStruct((NC, N, H), jnp.float32),
      mesh=_mesh(),
      scratch_types=[
          pltpu.VMEM((N_CHUNKS, CHUNK), jnp.int32),
          pltpu.VMEM((BL, H), jnp.float32),
          pltpu.VMEM((BL, H), jnp.float32),
          pltpu.VMEM_SHARED((N, H), jnp.float32),
          pltpu.SemaphoreType.DMA,
          pltpu.SemaphoreType.DMA,
          pltpu.SemaphoreType.DMA,
          pltpu.SemaphoreType.DMA,
      ],
  )
  def k(m_hbm, dst_hbm, z_hbm, out_hbm, idxb, v0, v1, acc, l0, l1, a0, a1):
    c = lax.axis_index("c")
    s = lax.axis_index("s")
    w2 = c * NS + s
    rsl = pl.ds(s * ROWS_PER_SUB, ROWS_PER_SUB)
    tsl = pl.ds(NS * ROWS_PER_SUB, TAIL_ROWS)
    pltpu.sync_copy(z_hbm, acc.at[rsl])

    @pl.when(s == 0)
    def _():
      pltpu.sync_copy(z_hbm.at[pl.ds(0, TAIL_ROWS)], acc.at[tsl])

    pltpu.sync_copy(dst_hbm.at[w2], idxb)
    plsc.subcore_barrier()
    ebase = w2 * E_PER_W
    vv = (v0, v1)
    ls = (l0, l1)
    asem = (a0, a1)

    def fire_load(j, b):
      pltpu.async_copy(m_hbm.at[pl.ds(ebase + j * BL, BL)], vv[b], ls[b])

    def drain_load(b):
      pltpu.make_async_copy(m_hbm.at[pl.ds(ebase, BL)], vv[b], ls[b]).wait()

    def fire_adds(j, b):
      for t in range(SUB):
        pltpu.async_copy(vv[b].at[pl.ds(t * CHUNK, CHUNK)],
                         acc.at[idxb.at[j * SUB + t]], asem[b], add=True)

    def drain_adds(b):
      for t in range(SUB):
        pltpu.make_async_copy(vv[b].at[pl.ds(t * CHUNK, CHUNK)],
                              acc.at[idxb.at[0]], asem[b]).wait()

    fire_load(0, 0)

    def pair(i, carry):
      j0 = 2 * i

      @pl.when(i > 0)
      def _():
        drain_adds(1)

      fire_load(j0 + 1, 1)
      drain_load(0)
      fire_adds(j0, 0)

      @pl.when(j0 + 2 < N_BLOCKS)
      def _():
        drain_adds(0)
        fire_load(j0 + 2, 0)

      drain_load(1)
      fire_adds(j0 + 1, 1)
      return carry

    lax.fori_loop(0, N_BLOCKS // 2, pair, 0)
    drain_adds(1)
    drain_load(0)
    fire_adds(N_BLOCKS - 1, 0)
    drain_adds(0)
    plsc.subcore_barrier()
    pltpu.sync_copy(acc.at[rsl], out_hbm.at[c].at[rsl])

    @pl.when(s == 0)
    def _():
      pltpu.sync_copy(acc.at[tsl], out_hbm.at[c].at[tsl])

  return k(m, dst3d, zrows)


def _counts(dst3d, zrows):
  @functools.partial(
      pl.kernel,
      out_type=jax.ShapeDtypeStruct((NC, N, CW), jnp.float32),
      mesh=_mesh(),
      scratch_types=[
          pltpu.VMEM((N_CHUNKS, CHUNK), jnp.int32),
          pltpu.VMEM((CHUNK, CW), jnp.float32),
          pltpu.VMEM_SHARED((N, CW), jnp.float32),
          pltpu.SemaphoreType.DMA,
      ],
  )
  def k(dst_hbm, z_hbm, out_hbm, idxb, ones_v, acc, asem):
    c = lax.axis_index("c")
    s = lax.axis_index("s")
    w2 = c * NS + s
    rsl = pl.ds(s * ROWS_PER_SUB, ROWS_PER_SUB)
    tsl = pl.ds(NS * ROWS_PER_SUB, TAIL_ROWS)
    pltpu.sync_copy(z_hbm, acc.at[rsl])

    @pl.when(s == 0)
    def _():
      pltpu.sync_copy(z_hbm.at[pl.ds(0, TAIL_ROWS)], acc.at[tsl])

    pltpu.sync_copy(dst_hbm.at[w2], idxb)
    one = jnp.ones((16,), jnp.float32)

    def fill_row(r, carry):
      for cc in range(CW // 16):
        ones_v[r, pl.ds(cc * 16, 16)] = one
      return carry

    lax.fori_loop(0, CHUNK, fill_row, 0)
    plsc.subcore_barrier()

    def block(j, carry):
      pltpu.async_copy(ones_v, acc.at[idxb.at[2 * j]], asem, add=True)
      pltpu.async_copy(ones_v, acc.at[idxb.at[2 * j + 1]], asem, add=True)
      pltpu.make_async_copy(ones_v, acc.at[idxb.at[0]], asem).wait()
      pltpu.make_async_copy(ones_v, acc.at[idxb.at[0]], asem).wait()
      return carry

    lax.fori_loop(0, N_CHUNKS // 2, block, 0)
    pltpu.sync_copy(ones_v, acc.at[idxb.at[N_CHUNKS - 1]], add=True)
    plsc.subcore_barrier()
    pltpu.sync_copy(acc.at[rsl], out_hbm.at[c].at[rsl])

    @pl.when(s == 0)
    def _():
      pltpu.sync_copy(acc.at[tsl], out_hbm.at[c].at[tsl])

  return k(dst3d, zrows)


def _edge_mlp(g, ef, w1c, b1, w2, b2):
  def body(g_ref, ef_ref, w1_ref, b1_ref, w2_ref, b2_ref, m_ref, efo_ref):
    efv = ef_ref[...]
    pre = g_ref[...] + jnp.dot(
        efv, w1_ref[...], preferred_element_type=jnp.float32) + b1_ref[...]
    h = jnp.maximum(pre, 0.0)
    m = jnp.dot(h, w2_ref[...],
                preferred_element_type=jnp.float32) + b2_ref[...]
    m_ref[...] = m
    efo_ref[...] = efv + m

  return pl.pallas_call(
      body,
      grid=(E // BE,),
      in_specs=[
          pl.BlockSpec((BE, H), lambda i: (i, 0)),
          pl.BlockSpec((BE, H), lambda i: (i, 0)),
          pl.BlockSpec((H, H), lambda i: (0, 0)),
          pl.BlockSpec((1, H), lambda i: (0, 0)),
          pl.BlockSpec((H, H), lambda i: (0, 0)),
          pl.BlockSpec((1, H), lambda i: (0, 0)),
      ],
      out_specs=[pl.BlockSpec((BE, H), lambda i: (i, 0))] * 2,
      out_shape=[jax.ShapeDtypeStruct((E, H), jnp.float32)] * 2,
  )(g, ef, w1c, b1, w2, b2)


def _edge_mlp0(ea_t, enc_w, enc_b, w1c, b1, w2, b2):
  def body(ea_ref, ew_ref, eb_ref, w1_ref, b1_ref, w2_ref, b2_ref, m_ref,
           efo_ref):
    ef = lax.dot_general(
        ea_ref[...], ew_ref[...], (((0,), (0,)), ((), ())),
        preferred_element_type=jnp.float32) + eb_ref[...]
    pre = jnp.dot(ef, w1_ref[...],
                  preferred_element_type=jnp.float32) + b1_ref[...]
    h = jnp.maximum(pre, 0.0)
    m = jnp.dot(h, w2_ref[...],
                preferred_element_type=jnp.float32) + b2_ref[...]
    m_ref[...] = m
    efo_ref[...] = ef + m

  return pl.pallas_call(
      body,
      grid=(E // BE,),
      in_specs=[
          pl.BlockSpec((3, BE), lambda i: (0, i)),
          pl.BlockSpec((3, H), lambda i: (0, 0)),
          pl.BlockSpec((1, H), lambda i: (0, 0)),
          pl.BlockSpec((H, H), lambda i: (0, 0)),
          pl.BlockSpec((1, H), lambda i: (0, 0)),
          pl.BlockSpec((H, H), lambda i: (0, 0)),
          pl.BlockSpec((1, H), lambda i: (0, 0)),
      ],
      out_specs=[pl.BlockSpec((BE, H), lambda i: (i, 0))] * 2,
      out_shape=[jax.ShapeDtypeStruct((E, H), jnp.float32)] * 2,
  )(ea_t, enc_w, enc_b, w1c, b1, w2, b2)


def _node_mlp(x, p, cnts, wa, wb, b1, w2, b2, w1a_n, w1b_n):
  with_next = w1a_n is not None

  def body(x_ref, p_ref, c_ref, wa_ref, wb_ref, b1_ref, w2_ref, b2_ref,
           *rest):
    cnt = c_ref[0, :, 0:1] + c_ref[1, :, 0:1]
    inv = 1.0 / jnp.maximum(cnt, 1.0)
    aggr = (p_ref[0] + p_ref[1]) * inv
    xv = x_ref[...]
    h = jnp.maximum(
        jnp.dot(xv, wa_ref[...], preferred_element_type=jnp.float32)
        + jnp.dot(aggr, wb_ref[...], preferred_element_type=jnp.float32)
        + b1_ref[...], 0.0)
    xo = xv + jnp.dot(h, w2_ref[...],
                      preferred_element_type=jnp.float32) + b2_ref[...]
    if with_next:
      w1a_ref, w1b_ref, xo_ref, x1_ref, x2_ref = rest
      xo_ref[...] = xo
      x1_ref[...] = jnp.dot(xo, w1a_ref[...],
                            preferred_element_type=jnp.float32)
      x2_ref[...] = jnp.dot(xo, w1b_ref[...],
                            preferred_element_type=jnp.float32)
    else:
      rest[0][...] = xo

  in_specs = [
      pl.BlockSpec((BN, H), lambda i: (i, 0)),
      pl.BlockSpec((NC, BN, H), lambda i: (0, i, 0)),
      pl.BlockSpec((NC, BN, CW), lambda i: (0, i, 0)),
      pl.BlockSpec((H, H), lambda i: (0, 0)),
      pl.BlockSpec((H, H), lambda i: (0, 0)),
      pl.BlockSpec((1, H), lambda i: (0, 0)),
      pl.BlockSpec((H, H), lambda i: (0, 0)),
      pl.BlockSpec((1, H), lambda i: (0, 0)),
  ]
  args = [x, p, cnts, wa, wb, b1, w2, b2]
  n_out = 1
  if with_next:
    in_specs += [pl.BlockSpec((H, H), lambda i: (0, 0))] * 2
    args += [w1a_n, w1b_n]
    n_out = 3
  out = pl.pallas_call(
      body,
      grid=(N // BN,),
      in_specs=in_specs,
      out_specs=[pl.BlockSpec((BN, H), lambda i: (i, 0))] * n_out,
      out_shape=[jax.ShapeDtypeStruct((N, H), jnp.float32)] * n_out,
  )(*args)
  return out


def _decode(x, w_pad, b_pad):
  def body(x_ref, w_ref, b_ref, o_ref):
    out = jnp.dot(x_ref[...], w_ref[...],
                  preferred_element_type=jnp.float32) + b_ref[...]
    ss = jnp.sum(out * out, axis=1, keepdims=True)
    o_ref[...] = out / jnp.maximum(jnp.sqrt(ss), 1e-12)

  return pl.pallas_call(
      body,
      grid=(N // BN,),
      in_specs=[
          pl.BlockSpec((BN, H), lambda i: (i, 0)),
          pl.BlockSpec((H, H), lambda i: (0, 0)),
          pl.BlockSpec((1, H), lambda i: (0, 0)),
      ],
      out_specs=pl.BlockSpec((BN, H), lambda i: (i, 0)),
      out_shape=jax.ShapeDtypeStruct((N, H), jnp.float32),
  )(x, w_pad, b_pad)


def kernel(pos, edge_attr, edge_index, enc_W, enc_b, dec_W, dec_b, e_W1,
           e_b1, e_W2, e_b2, n_W1, n_b1, n_W2, n_b2):
  del pos
  f32 = jnp.float32
  src = edge_index[0]
  dst = edge_index[1]
  dst3d = dst.reshape(NW, N_CHUNKS, CHUNK)
  ea_t = edge_attr.T
  enc_b2 = enc_b.reshape(1, H)
  zrows = jnp.zeros((ROWS_PER_SUB, H), f32)
  dec_w_pad = jnp.zeros((H, H), f32).at[:, :3].set(dec_W)
  dec_b_pad = jnp.zeros((1, H), f32).at[0, :3].set(dec_b)

  cnts = _counts(dst3d, zrows)

  x = jnp.zeros((N, H), f32)
  ef = None
  g = None
  for i in range(NLAYERS):
    w1c = e_W1[i, 2 * H:3 * H]
    b1 = e_b1[i].reshape(1, H)
    w2 = e_W2[i]
    b2 = e_b2[i].reshape(1, H)
    if i == 0:
      m, ef = _edge_mlp0(ea_t, enc_W, enc_b2, w1c, b1, w2, b2)
    else:
      m, ef = _edge_mlp(g, ef, w1c, b1, w2, b2)
    p = _scatter_add(m, dst3d, zrows)
    wa = n_W1[i, :H]
    wb = n_W1[i, H:]
    nb1 = n_b1[i].reshape(1, H)
    nw2 = n_W2[i]
    nb2 = n_b2[i].reshape(1, H)
    if i < NLAYERS - 1:
      x, x1t, x2t = _node_mlp(x, p, cnts, wa, wb, nb1, nw2, nb2,
                              e_W1[i + 1, :H], e_W1[i + 1, H:2 * H])
      g = _gather_sum(x1t, x2t, dst, src)
    else:
      (x,) = _node_mlp(x, p, cnts, wa, wb, nb1, nw2, nb2, None, None)
  out = _decode(x, dec_w_pad, dec_b_pad)
  return out[:, :3]


# speedup vs baseline: 4.1367x; 1.0002x over previous
import functools

import jax
import jax.numpy as jnp
from jax import lax
from jax.experimental import pallas as pl
from jax.experimental.pallas import tpu as pltpu
from jax.experimental.pallas import tpu_sc as plsc

H = 128
N = 10000
E = 320000
NLAYERS = 6

NC = 2
NS = 16
NW = NC * NS
E_PER_W = E // NW
CHUNK = 80
N_CHUNKS = E_PER_W // CHUNK
BL = 80
N_BLOCKS = E_PER_W // BL
ROWS_PER_SUB = 624
TAIL_ROWS = N - NS * ROWS_PER_SUB
CW = 128

BE = 1280
BN = 2000

_mesh = functools.partial(
    plsc.VectorSubcoreMesh, core_axis_name="c", subcore_axis_name="s"
)


def _gather_sum(x1, x2, dst, src):
  @functools.partial(
      pl.kernel,
      out_type=jax.ShapeDtypeStruct((E, H), jnp.float32),
      mesh=_mesh(),
      scratch_types=[
          pltpu.VMEM((E_PER_W,), jnp.int32),
          pltpu.VMEM((E_PER_W,), jnp.int32),
          pltpu.VMEM((CHUNK, H), jnp.float32),
          pltpu.VMEM((CHUNK, H), jnp.float32),
          pltpu.VMEM((CHUNK, H), jnp.float32),
          pltpu.VMEM((CHUNK, H), jnp.float32),
          pltpu.SemaphoreType.DMA,
          pltpu.SemaphoreType.DMA,
          pltpu.SemaphoreType.DMA,
          pltpu.SemaphoreType.DMA,
      ],
  )
  def k(x1_hbm, x2_hbm, dst_hbm, src_hbm, out_hbm, idx_d, idx_s, ra0, rb0,
        ra1, rb1, g0, g1, s0, s1):
    w = lax.axis_index("s") * NC + lax.axis_index("c")
    base = w * E_PER_W
    pltpu.sync_copy(dst_hbm.at[pl.ds(base, E_PER_W)], idx_d)
    pltpu.sync_copy(src_hbm.at[pl.ds(base, E_PER_W)], idx_s)
    ra = (ra0, ra1)
    rb = (rb0, rb1)
    gs = (g0, g1)
    ss = (s0, s1)

    def fire(kk, b):
      isl = pl.ds(kk * CHUNK, CHUNK)
      pltpu.async_copy(x1_hbm.at[idx_d.at[isl]], ra[b], gs[b])
      pltpu.async_copy(x2_hbm.at[idx_s.at[isl]], rb[b], gs[b])

    def drain_add(kk, b):
      pltpu.make_async_copy(x1_hbm.at[idx_d.at[pl.ds(0, CHUNK)]], ra[b],
                            gs[b]).wait()
      pltpu.make_async_copy(x2_hbm.at[idx_s.at[pl.ds(0, CHUNK)]], rb[b],
                            gs[b]).wait()

      def add_row(r, c2):
        for cc in range(H // 16):
          sl = pl.ds(cc * 16, 16)
          plsc.addupdate(ra[b].at[r, sl], rb[b][r, sl])
        return c2

      lax.fori_loop(0, CHUNK, add_row, 0)
      pltpu.async_copy(ra[b], out_hbm.at[pl.ds(base + kk * CHUNK, CHUNK)],
                       ss[b])

    def wait_store(b):
      pltpu.make_async_copy(ra[b], out_hbm.at[pl.ds(base, CHUNK)],
                            ss[b]).wait()

    fire(0, 0)

    def pair(i, carry):
      k0 = 2 * i

      @pl.when(i > 0)
      def _():
        wait_store(1)

      fire(k0 + 1, 1)
      drain_add(k0, 0)

      @pl.when(k0 + 2 < N_CHUNKS)
      def _():
        wait_store(0)
        fire(k0 + 2, 0)

      drain_add(k0 + 1, 1)
      return carry

    lax.fori_loop(0, N_CHUNKS // 2, pair, 0)
    drain_add(N_CHUNKS - 1, 0)
    wait_store(0)
    wait_store(1)

  return k(x1, x2, dst, src)


def _scatter_add(m, dst3d, zrows):
  SUB = BL // CHUNK

  @functools.partial(
      pl.kernel,
      out_type=jax.ShapeDtypeStruct((NC, N, H), jnp.float32),
      mesh=_mesh(),
      scratch_types=[
          pltpu.VMEM((N_CHUNKS, CHUNK), jnp.int32),
          pltpu.VMEM((BL, H), jnp.float32),
          pltpu.VMEM((BL, H), jnp.float32),
          pltpu.VMEM_SHARED((N, H), jnp.float32),
          pltpu.SemaphoreType.DMA,
          pltpu.SemaphoreType.DMA,
          pltpu.SemaphoreType.DMA,
          pltpu.SemaphoreType.DMA,
      ],
  )
  def k(m_hbm, dst_hbm, z_hbm, out_hbm, idxb, v0, v1, acc, l0, l1, a0, a1):
    c = lax.axis_index("c")
    s = lax.axis_index("s")
    w2 = c * NS + s
    rsl = pl.ds(s * ROWS_PER_SUB, ROWS_PER_SUB)
    tsl = pl.ds(NS * ROWS_PER_SUB, TAIL_ROWS)
    pltpu.sync_copy(z_hbm, acc.at[rsl])

    @pl.when(s == 0)
    def _():
      pltpu.sync_copy(z_hbm.at[pl.ds(0, TAIL_ROWS)], acc.at[tsl])

    pltpu.sync_copy(dst_hbm.at[w2], idxb)
    plsc.subcore_barrier()
    ebase = w2 * E_PER_W
    vv = (v0, v1)
    ls = (l0, l1)
    asem = (a0, a1)

    def fire_load(j, b):
      pltpu.async_copy(m_hbm.at[pl.ds(ebase + j * BL, BL)], vv[b], ls[b])

    def drain_load(b):
      pltpu.make_async_copy(m_hbm.at[pl.ds(ebase, BL)], vv[b], ls[b]).wait()

    def fire_adds(j, b):
      for t in range(SUB):
        pltpu.async_copy(vv[b].at[pl.ds(t * CHUNK, CHUNK)],
                         acc.at[idxb.at[j * SUB + t]], asem[b], add=True)

    def drain_adds(b):
      for t in range(SUB):
        pltpu.make_async_copy(vv[b].at[pl.ds(t * CHUNK, CHUNK)],
                              acc.at[idxb.at[0]], asem[b]).wait()

    fire_load(0, 0)

    def pair(i, carry):
      j0 = 2 * i

      @pl.when(i > 0)
      def _():
        drain_adds(1)

      fire_load(j0 + 1, 1)
      drain_load(0)
      fire_adds(j0, 0)

      @pl.when(j0 + 2 < N_BLOCKS)
      def _():
        drain_adds(0)
        fire_load(j0 + 2, 0)

      drain_load(1)
      fire_adds(j0 + 1, 1)
      return carry

    lax.fori_loop(0, N_BLOCKS // 2, pair, 0)
    drain_adds(1)
    drain_load(0)
    fire_adds(N_BLOCKS - 1, 0)
    drain_adds(0)
    plsc.subcore_barrier()
    pltpu.sync_copy(acc.at[rsl], out_hbm.at[c].at[rsl])

    @pl.when(s == 0)
    def _():
      pltpu.sync_copy(acc.at[tsl], out_hbm.at[c].at[tsl])

  return k(m, dst3d, zrows)


def _counts(dst3d, zrows):
  @functools.partial(
      pl.kernel,
      out_type=jax.ShapeDtypeStruct((NC, N, CW), jnp.float32),
      mesh=_mesh(),
      scratch_types=[
          pltpu.VMEM((N_CHUNKS, CHUNK), jnp.int32),
          pltpu.VMEM((CHUNK, CW), jnp.float32),
          pltpu.VMEM_SHARED((N, CW), jnp.float32),
          pltpu.SemaphoreType.DMA,
      ],
  )
  def k(dst_hbm, z_hbm, out_hbm, idxb, ones_v, acc, asem):
    c = lax.axis_index("c")
    s = lax.axis_index("s")
    w2 = c * NS + s
    rsl = pl.ds(s * ROWS_PER_SUB, ROWS_PER_SUB)
    tsl = pl.ds(NS * ROWS_PER_SUB, TAIL_ROWS)
    pltpu.sync_copy(z_hbm, acc.at[rsl])

    @pl.when(s == 0)
    def _():
      pltpu.sync_copy(z_hbm.at[pl.ds(0, TAIL_ROWS)], acc.at[tsl])

    pltpu.sync_copy(dst_hbm.at[w2], idxb)
    one = jnp.ones((16,), jnp.float32)

    def fill_row(r, carry):
      for cc in range(CW // 16):
        ones_v[r, pl.ds(cc * 16, 16)] = one
      return carry

    lax.fori_loop(0, CHUNK, fill_row, 0)
    plsc.subcore_barrier()

    def block(j, carry):
      pltpu.async_copy(ones_v, acc.at[idxb.at[2 * j]], asem, add=True)
      pltpu.async_copy(ones_v, acc.at[idxb.at[2 * j + 1]], asem, add=True)
      pltpu.make_async_copy(ones_v, acc.at[idxb.at[0]], asem).wait()
      pltpu.make_async_copy(ones_v, acc.at[idxb.at[0]], asem).wait()
      return carry

    lax.fori_loop(0, N_CHUNKS // 2, block, 0)
    pltpu.sync_copy(ones_v, acc.at[idxb.at[N_CHUNKS - 1]], add=True)
    plsc.subcore_barrier()
    pltpu.sync_copy(acc.at[rsl], out_hbm.at[c].at[rsl])

    @pl.when(s == 0)
    def _():
      pltpu.sync_copy(acc.at[tsl], out_hbm.at[c].at[tsl])

  return k(dst3d, zrows)


def _edge_mlp(g, ef, w1c, b1, w2, b2):
  def body(g_ref, ef_ref, w1_ref, b1_ref, w2_ref, b2_ref, m_ref, efo_ref):
    efv = ef_ref[...]
    pre = g_ref[...] + jnp.dot(
        efv, w1_ref[...], preferred_element_type=jnp.float32) + b1_ref[...]
    h = jnp.maximum(pre, 0.0)
    m = jnp.dot(h, w2_ref[...],
                preferred_element_type=jnp.float32) + b2_ref[...]
    m_ref[...] = m
    efo_ref[...] = efv + m

  return pl.pallas_call(
      body,
      grid=(E // BE,),
      in_specs=[
          pl.BlockSpec((BE, H), lambda i: (i, 0)),
          pl.BlockSpec((BE, H), lambda i: (i, 0)),
          pl.BlockSpec((H, H), lambda i: (0, 0)),
          pl.BlockSpec((1, H), lambda i: (0, 0)),
          pl.BlockSpec((H, H), lambda i: (0, 0)),
          pl.BlockSpec((1, H), lambda i: (0, 0)),
      ],
      out_specs=[pl.BlockSpec((BE, H), lambda i: (i, 0))] * 2,
      out_shape=[jax.ShapeDtypeStruct((E, H), jnp.float32)] * 2,
  )(g, ef, w1c, b1, w2, b2)


def _edge_mlp0(ea_t, enc_w, enc_b, w1c, b1, w2, b2):
  def body(ea_ref, ew_ref, eb_ref, w1_ref, b1_ref, w2_ref, b2_ref, m_ref,
           efo_ref):
    ef = lax.dot_general(
        ea_ref[...], ew_ref[...], (((0,), (0,)), ((), ())),
        preferred_element_type=jnp.float32) + eb_ref[...]
    pre = jnp.dot(ef, w1_ref[...],
                  preferred_element_type=jnp.float32) + b1_ref[...]
    h = jnp.maximum(pre, 0.0)
    m = jnp.dot(h, w2_ref[...],
                preferred_element_type=jnp.float32) + b2_ref[...]
    m_ref[...] = m
    efo_ref[...] = ef + m

  return pl.pallas_call(
      body,
      grid=(E // BE,),
      in_specs=[
          pl.BlockSpec((3, BE), lambda i: (0, i)),
          pl.BlockSpec((3, H), lambda i: (0, 0)),
          pl.BlockSpec((1, H), lambda i: (0, 0)),
          pl.BlockSpec((H, H), lambda i: (0, 0)),
          pl.BlockSpec((1, H), lambda i: (0, 0)),
          pl.BlockSpec((H, H), lambda i: (0, 0)),
          pl.BlockSpec((1, H), lambda i: (0, 0)),
      ],
      out_specs=[pl.BlockSpec((BE, H), lambda i: (i, 0))] * 2,
      out_shape=[jax.ShapeDtypeStruct((E, H), jnp.float32)] * 2,
  )(ea_t, enc_w, enc_b, w1c, b1, w2, b2)


def _node_mlp(x, p, cnts, wa, wb, b1, w2, b2, w1a_n, w1b_n):
  with_next = w1a_n is not None

  def body(x_ref, p_ref, c_ref, wa_ref, wb_ref, b1_ref, w2_ref, b2_ref,
           *rest):
    cnt = c_ref[0, :, 0:1] + c_ref[1, :, 0:1]
    aggr = (p_ref[0] + p_ref[1]) / jnp.maximum(cnt, 1.0)
    xv = x_ref[...]
    h = jnp.maximum(
        jnp.dot(xv, wa_ref[...], preferred_element_type=jnp.float32)
        + jnp.dot(aggr, wb_ref[...], preferred_element_type=jnp.float32)
        + b1_ref[...], 0.0)
    xo = xv + jnp.dot(h, w2_ref[...],
                      preferred_element_type=jnp.float32) + b2_ref[...]
    if with_next:
      w1a_ref, w1b_ref, xo_ref, x1_ref, x2_ref = rest
      xo_ref[...] = xo
      x1_ref[...] = jnp.dot(xo, w1a_ref[...],
                            preferred_element_type=jnp.float32)
      x2_ref[...] = jnp.dot(xo, w1b_ref[...],
                            preferred_element_type=jnp.float32)
    else:
      rest[0][...] = xo

  in_specs = [
      pl.BlockSpec((BN, H), lambda i: (i, 0)),
      pl.BlockSpec((NC, BN, H), lambda i: (0, i, 0)),
      pl.BlockSpec((NC, BN, CW), lambda i: (0, i, 0)),
      pl.BlockSpec((H, H), lambda i: (0, 0)),
      pl.BlockSpec((H, H), lambda i: (0, 0)),
      pl.BlockSpec((1, H), lambda i: (0, 0)),
      pl.BlockSpec((H, H), lambda i: (0, 0)),
      pl.BlockSpec((1, H), lambda i: (0, 0)),
  ]
  args = [x, p, cnts, wa, wb, b1, w2, b2]
  n_out = 1
  if with_next:
    in_specs += [pl.BlockSpec((H, H), lambda i: (0, 0))] * 2
    args += [w1a_n, w1b_n]
    n_out = 3
  out = pl.pallas_call(
      body,
      grid=(N // BN,),
      in_specs=in_specs,
      out_specs=[pl.BlockSpec((BN, H), lambda i: (i, 0))] * n_out,
      out_shape=[jax.ShapeDtypeStruct((N, H), jnp.float32)] * n_out,
  )(*args)
  return out


def _decode(x, w_pad, b_pad):
  def body(x_ref, w_ref, b_ref, o_ref):
    out = jnp.dot(x_ref[...], w_ref[...],
                  preferred_element_type=jnp.float32) + b_ref[...]
    ss = jnp.sum(out * out, axis=1, keepdims=True)
    o_ref[...] = out / jnp.maximum(jnp.sqrt(ss), 1e-12)

  return pl.pallas_call(
      body,
      grid=(N // BN,),
      in_specs=[
          pl.BlockSpec((BN, H), lambda i: (i, 0)),
          pl.BlockSpec((H, H), lambda i: (0, 0)),
          pl.BlockSpec((1, H), lambda i: (0, 0)),
      ],
      out_specs=pl.BlockSpec((BN, H), lambda i: (i, 0)),
      out_shape=jax.ShapeDtypeStruct((N, H), jnp.float32),
  )(x, w_pad, b_pad)


def kernel(pos, edge_attr, edge_index, enc_W, enc_b, dec_W, dec_b, e_W1,
           e_b1, e_W2, e_b2, n_W1, n_b1, n_W2, n_b2):
  del pos
  f32 = jnp.float32
  src = edge_index[0]
  dst = edge_index[1]
  dst3d = dst.reshape(NW, N_CHUNKS, CHUNK)
  ea_t = edge_attr.T
  enc_b2 = enc_b.reshape(1, H)
  zrows = jnp.zeros((ROWS_PER_SUB, H), f32)
  dec_w_pad = jnp.zeros((H, H), f32).at[:, :3].set(dec_W)
  dec_b_pad = jnp.zeros((1, H), f32).at[0, :3].set(dec_b)

  cnts = _counts(dst3d, zrows)

  x = jnp.zeros((N, H), f32)
  ef = None
  g = None
  for i in range(NLAYERS):
    b1 = e_b1[i].reshape(1, H)
    w2 = e_W2[i]
    b2 = e_b2[i].reshape(1, H)
    if i == 0:
      m, ef = _edge_mlp0(ea_t, enc_W, enc_b2, e_W1[0, 2 * H:3 * H], b1, w2,
                         b2)
    else:
      m, ef = _edge_mlp(g, ef, e_W1[i, 2 * H:3 * H], b1, w2, b2)
    p = _scatter_add(m, dst3d, zrows)
    nb1 = n_b1[i].reshape(1, H)
    nb2 = n_b2[i].reshape(1, H)
    if i < NLAYERS - 1:
      x, x1t, x2t = _node_mlp(x, p, cnts, n_W1[i, :H], n_W1[i, H:], nb1,
                              n_W2[i], nb2, e_W1[i + 1, :H],
                              e_W1[i + 1, H:2 * H])
      g = _gather_sum(x1t, x2t, dst, src)
    else:
      (x,) = _node_mlp(x, p, cnts, n_W1[i, :H], n_W1[i, H:], nb1, n_W2[i],
                       nb2, None, None)
  out = _decode(x, dec_w_pad, dec_b_pad)
  return out[:, :3]


# speedup vs baseline: 4.3674x; 1.0558x over previous
import functools

import jax
import jax.numpy as jnp
from jax import lax
from jax.experimental import pallas as pl
from jax.experimental.pallas import tpu as pltpu
from jax.experimental.pallas import tpu_sc as plsc

H = 128
N = 10000
E = 320000
NLAYERS = 6

NC = 2
NS = 16
NW = NC * NS
E_PER_W = E // NW
CHUNK = 80
N_CHUNKS = E_PER_W // CHUNK
BL = 80
N_BLOCKS = E_PER_W // BL
ROWS_PER_SUB = 624
TAIL_ROWS = N - NS * ROWS_PER_SUB
CW = 128

BE = 1280
BN = 2000

_mesh = functools.partial(
    plsc.VectorSubcoreMesh, core_axis_name="c", subcore_axis_name="s"
)


def _gather_sum(x1, x2, dst, src):
  EH = dst.shape[0]
  E_PER_W = EH // NW
  CHUNK = E_PER_W // N_CHUNKS

  @functools.partial(
      pl.kernel,
      out_type=jax.ShapeDtypeStruct((EH, H), jnp.float32),
      mesh=_mesh(),
      scratch_types=[
          pltpu.VMEM((E_PER_W,), jnp.int32),
          pltpu.VMEM((E_PER_W,), jnp.int32),
          pltpu.VMEM((CHUNK, H), jnp.float32),
          pltpu.VMEM((CHUNK, H), jnp.float32),
          pltpu.VMEM((CHUNK, H), jnp.float32),
          pltpu.VMEM((CHUNK, H), jnp.float32),
          pltpu.SemaphoreType.DMA,
          pltpu.SemaphoreType.DMA,
          pltpu.SemaphoreType.DMA,
          pltpu.SemaphoreType.DMA,
      ],
  )
  def k(x1_hbm, x2_hbm, dst_hbm, src_hbm, out_hbm, idx_d, idx_s, ra0, rb0,
        ra1, rb1, g0, g1, s0, s1):
    w = lax.axis_index("s") * NC + lax.axis_index("c")
    base = w * E_PER_W
    pltpu.sync_copy(dst_hbm.at[pl.ds(base, E_PER_W)], idx_d)
    pltpu.sync_copy(src_hbm.at[pl.ds(base, E_PER_W)], idx_s)
    ra = (ra0, ra1)
    rb = (rb0, rb1)
    gs = (g0, g1)
    ss = (s0, s1)

    def fire(kk, b):
      isl = pl.ds(kk * CHUNK, CHUNK)
      pltpu.async_copy(x1_hbm.at[idx_d.at[isl]], ra[b], gs[b])
      pltpu.async_copy(x2_hbm.at[idx_s.at[isl]], rb[b], gs[b])

    def drain_add(kk, b):
      pltpu.make_async_copy(x1_hbm.at[idx_d.at[pl.ds(0, CHUNK)]], ra[b],
                            gs[b]).wait()
      pltpu.make_async_copy(x2_hbm.at[idx_s.at[pl.ds(0, CHUNK)]], rb[b],
                            gs[b]).wait()

      def add_row(r, c2):
        for cc in range(H // 16):
          sl = pl.ds(cc * 16, 16)
          plsc.addupdate(ra[b].at[r, sl], rb[b][r, sl])
        return c2

      lax.fori_loop(0, CHUNK, add_row, 0)
      pltpu.async_copy(ra[b], out_hbm.at[pl.ds(base + kk * CHUNK, CHUNK)],
                       ss[b])

    def wait_store(b):
      pltpu.make_async_copy(ra[b], out_hbm.at[pl.ds(base, CHUNK)],
                            ss[b]).wait()

    fire(0, 0)

    def pair(i, carry):
      k0 = 2 * i

      @pl.when(i > 0)
      def _():
        wait_store(1)

      fire(k0 + 1, 1)
      drain_add(k0, 0)

      @pl.when(k0 + 2 < N_CHUNKS)
      def _():
        wait_store(0)
        fire(k0 + 2, 0)

      drain_add(k0 + 1, 1)
      return carry

    lax.fori_loop(0, N_CHUNKS // 2, pair, 0)
    drain_add(N_CHUNKS - 1, 0)
    wait_store(0)
    wait_store(1)

  return k(x1, x2, dst, src)


def _scatter_add(m, dst3d, zrows):
  EH = m.shape[0]
  E_PER_W = EH // NW
  CHUNK = E_PER_W // N_CHUNKS
  BL = CHUNK
  SUB = 1

  @functools.partial(
      pl.kernel,
      out_type=jax.ShapeDtypeStruct((NC, N, H), jnp.float32),
      mesh=_mesh(),
      scratch_types=[
          pltpu.VMEM((N_CHUNKS, CHUNK), jnp.int32),
          pltpu.VMEM((BL, H), jnp.float32),
          pltpu.VMEM((BL, H), jnp.float32),
          pltpu.VMEM_SHARED((N, H), jnp.float32),
          pltpu.SemaphoreType.DMA,
          pltpu.SemaphoreType.DMA,
          pltpu.SemaphoreType.DMA,
          pltpu.SemaphoreType.DMA,
      ],
  )
  def k(m_hbm, dst_hbm, z_hbm, out_hbm, idxb, v0, v1, acc, l0, l1, a0, a1):
    c = lax.axis_index("c")
    s = lax.axis_index("s")
    w2 = c * NS + s
    rsl = pl.ds(s * ROWS_PER_SUB, ROWS_PER_SUB)
    tsl = pl.ds(NS * ROWS_PER_SUB, TAIL_ROWS)
    pltpu.sync_copy(z_hbm, acc.at[rsl])

    @pl.when(s == 0)
    def _():
      pltpu.sync_copy(z_hbm.at[pl.ds(0, TAIL_ROWS)], acc.at[tsl])

    pltpu.sync_copy(dst_hbm.at[w2], idxb)
    plsc.subcore_barrier()
    ebase = w2 * E_PER_W
    vv = (v0, v1)
    ls = (l0, l1)
    asem = (a0, a1)

    def fire_load(j, b):
      pltpu.async_copy(m_hbm.at[pl.ds(ebase + j * BL, BL)], vv[b], ls[b])

    def drain_load(b):
      pltpu.make_async_copy(m_hbm.at[pl.ds(ebase, BL)], vv[b], ls[b]).wait()

    def fire_adds(j, b):
      for t in range(SUB):
        pltpu.async_copy(vv[b].at[pl.ds(t * CHUNK, CHUNK)],
                         acc.at[idxb.at[j * SUB + t]], asem[b], add=True)

    def drain_adds(b):
      for t in range(SUB):
        pltpu.make_async_copy(vv[b].at[pl.ds(t * CHUNK, CHUNK)],
                              acc.at[idxb.at[0]], asem[b]).wait()

    fire_load(0, 0)

    def pair(i, carry):
      j0 = 2 * i

      @pl.when(i > 0)
      def _():
        drain_adds(1)

      fire_load(j0 + 1, 1)
      drain_load(0)
      fire_adds(j0, 0)

      @pl.when(j0 + 2 < N_BLOCKS)
      def _():
        drain_adds(0)
        fire_load(j0 + 2, 0)

      drain_load(1)
      fire_adds(j0 + 1, 1)
      return carry

    lax.fori_loop(0, N_BLOCKS // 2, pair, 0)
    drain_adds(1)
    drain_load(0)
    fire_adds(N_BLOCKS - 1, 0)
    drain_adds(0)
    plsc.subcore_barrier()
    pltpu.sync_copy(acc.at[rsl], out_hbm.at[c].at[rsl])

    @pl.when(s == 0)
    def _():
      pltpu.sync_copy(acc.at[tsl], out_hbm.at[c].at[tsl])

  return k(m, dst3d, zrows)


def _counts(dst3d, zrows):
  @functools.partial(
      pl.kernel,
      out_type=jax.ShapeDtypeStruct((NC, N, CW), jnp.float32),
      mesh=_mesh(),
      scratch_types=[
          pltpu.VMEM((N_CHUNKS, CHUNK), jnp.int32),
          pltpu.VMEM((CHUNK, CW), jnp.float32),
          pltpu.VMEM_SHARED((N, CW), jnp.float32),
          pltpu.SemaphoreType.DMA,
      ],
  )
  def k(dst_hbm, z_hbm, out_hbm, idxb, ones_v, acc, asem):
    c = lax.axis_index("c")
    s = lax.axis_index("s")
    w2 = c * NS + s
    rsl = pl.ds(s * ROWS_PER_SUB, ROWS_PER_SUB)
    tsl = pl.ds(NS * ROWS_PER_SUB, TAIL_ROWS)
    pltpu.sync_copy(z_hbm, acc.at[rsl])

    @pl.when(s == 0)
    def _():
      pltpu.sync_copy(z_hbm.at[pl.ds(0, TAIL_ROWS)], acc.at[tsl])

    pltpu.sync_copy(dst_hbm.at[w2], idxb)
    one = jnp.ones((16,), jnp.float32)

    def fill_row(r, carry):
      for cc in range(CW // 16):
        ones_v[r, pl.ds(cc * 16, 16)] = one
      return carry

    lax.fori_loop(0, CHUNK, fill_row, 0)
    plsc.subcore_barrier()

    def block(j, carry):
      pltpu.async_copy(ones_v, acc.at[idxb.at[2 * j]], asem, add=True)
      pltpu.async_copy(ones_v, acc.at[idxb.at[2 * j + 1]], asem, add=True)
      pltpu.make_async_copy(ones_v, acc.at[idxb.at[0]], asem).wait()
      pltpu.make_async_copy(ones_v, acc.at[idxb.at[0]], asem).wait()
      return carry

    lax.fori_loop(0, N_CHUNKS // 2, block, 0)
    pltpu.sync_copy(ones_v, acc.at[idxb.at[N_CHUNKS - 1]], add=True)
    plsc.subcore_barrier()
    pltpu.sync_copy(acc.at[rsl], out_hbm.at[c].at[rsl])

    @pl.when(s == 0)
    def _():
      pltpu.sync_copy(acc.at[tsl], out_hbm.at[c].at[tsl])

  return k(dst3d, zrows)


def _edge_mlp(g, ef, w1c, b1, w2, b2):
  def body(g_ref, ef_ref, w1_ref, b1_ref, w2_ref, b2_ref, m_ref, efo_ref):
    efv = ef_ref[...]
    pre = g_ref[...] + jnp.dot(
        efv, w1_ref[...], preferred_element_type=jnp.float32) + b1_ref[...]
    h = jnp.maximum(pre, 0.0)
    m = jnp.dot(h, w2_ref[...],
                preferred_element_type=jnp.float32) + b2_ref[...]
    m_ref[...] = m
    efo_ref[...] = efv + m

  EH = g.shape[0]
  return pl.pallas_call(
      body,
      grid=(EH // BE,),
      in_specs=[
          pl.BlockSpec((BE, H), lambda i: (i, 0)),
          pl.BlockSpec((BE, H), lambda i: (i, 0)),
          pl.BlockSpec((H, H), lambda i: (0, 0)),
          pl.BlockSpec((1, H), lambda i: (0, 0)),
          pl.BlockSpec((H, H), lambda i: (0, 0)),
          pl.BlockSpec((1, H), lambda i: (0, 0)),
      ],
      out_specs=[pl.BlockSpec((BE, H), lambda i: (i, 0))] * 2,
      out_shape=[jax.ShapeDtypeStruct((EH, H), jnp.float32)] * 2,
  )(g, ef, w1c, b1, w2, b2)


def _edge_mlp0(ea_t, enc_w, enc_b, w1c, b1, w2, b2):
  def body(ea_ref, ew_ref, eb_ref, w1_ref, b1_ref, w2_ref, b2_ref, m_ref,
           efo_ref):
    ef = lax.dot_general(
        ea_ref[...], ew_ref[...], (((0,), (0,)), ((), ())),
        preferred_element_type=jnp.float32) + eb_ref[...]
    pre = jnp.dot(ef, w1_ref[...],
                  preferred_element_type=jnp.float32) + b1_ref[...]
    h = jnp.maximum(pre, 0.0)
    m = jnp.dot(h, w2_ref[...],
                preferred_element_type=jnp.float32) + b2_ref[...]
    m_ref[...] = m
    efo_ref[...] = ef + m

  EH = ea_t.shape[1]
  return pl.pallas_call(
      body,
      grid=(EH // BE,),
      in_specs=[
          pl.BlockSpec((3, BE), lambda i: (0, i)),
          pl.BlockSpec((3, H), lambda i: (0, 0)),
          pl.BlockSpec((1, H), lambda i: (0, 0)),
          pl.BlockSpec((H, H), lambda i: (0, 0)),
          pl.BlockSpec((1, H), lambda i: (0, 0)),
          pl.BlockSpec((H, H), lambda i: (0, 0)),
          pl.BlockSpec((1, H), lambda i: (0, 0)),
      ],
      out_specs=[pl.BlockSpec((BE, H), lambda i: (i, 0))] * 2,
      out_shape=[jax.ShapeDtypeStruct((EH, H), jnp.float32)] * 2,
  )(ea_t, enc_w, enc_b, w1c, b1, w2, b2)


def _node_mlp(x, pa, pb, cnts, wa, wb, b1, w2, b2, w1a_n, w1b_n):
  with_next = w1a_n is not None

  def body(x_ref, pa_ref, pb_ref, c_ref, wa_ref, wb_ref, b1_ref, w2_ref,
           b2_ref, *rest):
    cnt = c_ref[0, :, 0:1] + c_ref[1, :, 0:1]
    summed = (pa_ref[0] + pa_ref[1]) + (pb_ref[0] + pb_ref[1])
    aggr = summed / jnp.maximum(cnt, 1.0)
    xv = x_ref[...]
    h = jnp.maximum(
        jnp.dot(xv, wa_ref[...], preferred_element_type=jnp.float32)
        + jnp.dot(aggr, wb_ref[...], preferred_element_type=jnp.float32)
        + b1_ref[...], 0.0)
    xo = xv + jnp.dot(h, w2_ref[...],
                      preferred_element_type=jnp.float32) + b2_ref[...]
    if with_next:
      w1a_ref, w1b_ref, xo_ref, x1_ref, x2_ref = rest
      xo_ref[...] = xo
      x1_ref[...] = jnp.dot(xo, w1a_ref[...],
                            preferred_element_type=jnp.float32)
      x2_ref[...] = jnp.dot(xo, w1b_ref[...],
                            preferred_element_type=jnp.float32)
    else:
      rest[0][...] = xo

  in_specs = [
      pl.BlockSpec((BN, H), lambda i: (i, 0)),
      pl.BlockSpec((NC, BN, H), lambda i: (0, i, 0)),
      pl.BlockSpec((NC, BN, H), lambda i: (0, i, 0)),
      pl.BlockSpec((NC, BN, CW), lambda i: (0, i, 0)),
      pl.BlockSpec((H, H), lambda i: (0, 0)),
      pl.BlockSpec((H, H), lambda i: (0, 0)),
      pl.BlockSpec((1, H), lambda i: (0, 0)),
      pl.BlockSpec((H, H), lambda i: (0, 0)),
      pl.BlockSpec((1, H), lambda i: (0, 0)),
  ]
  args = [x, pa, pb, cnts, wa, wb, b1, w2, b2]
  n_out = 1
  if with_next:
    in_specs += [pl.BlockSpec((H, H), lambda i: (0, 0))] * 2
    args += [w1a_n, w1b_n]
    n_out = 3
  out = pl.pallas_call(
      body,
      grid=(N // BN,),
      in_specs=in_specs,
      out_specs=[pl.BlockSpec((BN, H), lambda i: (i, 0))] * n_out,
      out_shape=[jax.ShapeDtypeStruct((N, H), jnp.float32)] * n_out,
  )(*args)
  return out


def _decode(x, w_pad, b_pad):
  def body(x_ref, w_ref, b_ref, o_ref):
    out = jnp.dot(x_ref[...], w_ref[...],
                  preferred_element_type=jnp.float32) + b_ref[...]
    ss = jnp.sum(out * out, axis=1, keepdims=True)
    o_ref[...] = out / jnp.maximum(jnp.sqrt(ss), 1e-12)

  return pl.pallas_call(
      body,
      grid=(N // BN,),
      in_specs=[
          pl.BlockSpec((BN, H), lambda i: (i, 0)),
          pl.BlockSpec((H, H), lambda i: (0, 0)),
          pl.BlockSpec((1, H), lambda i: (0, 0)),
      ],
      out_specs=pl.BlockSpec((BN, H), lambda i: (i, 0)),
      out_shape=jax.ShapeDtypeStruct((N, H), jnp.float32),
  )(x, w_pad, b_pad)


def kernel(pos, edge_attr, edge_index, enc_W, enc_b, dec_W, dec_b, e_W1,
           e_b1, e_W2, e_b2, n_W1, n_b1, n_W2, n_b2):
  del pos
  f32 = jnp.float32
  src = edge_index[0]
  dst = edge_index[1]
  dst3d = dst.reshape(NW, N_CHUNKS, CHUNK)
  E2 = E // 2
  CH2 = (E2 // NW) // N_CHUNKS
  dst_h = (dst[:E2], dst[E2:])
  src_h = (src[:E2], src[E2:])
  dst3d_h = (dst[:E2].reshape(NW, N_CHUNKS, CH2),
             dst[E2:].reshape(NW, N_CHUNKS, CH2))
  ea_t = edge_attr.T
  ea_t_h = (ea_t[:, :E2], ea_t[:, E2:])
  enc_b2 = enc_b.reshape(1, H)
  zrows = jnp.zeros((ROWS_PER_SUB, H), f32)
  dec_w_pad = jnp.zeros((H, H), f32).at[:, :3].set(dec_W)
  dec_b_pad = jnp.zeros((1, H), f32).at[0, :3].set(dec_b)

  cnts = _counts(dst3d, zrows)

  x = jnp.zeros((N, H), f32)
  ef = [None, None]
  g = [None, None]
  p = [None, None]
  for i in range(NLAYERS):
    b1 = e_b1[i].reshape(1, H)
    w2 = e_W2[i]
    b2 = e_b2[i].reshape(1, H)
    w1c = e_W1[i, 2 * H:3 * H]
    for hh in range(2):
      if i == 0:
        m, ef[hh] = _edge_mlp0(ea_t_h[hh], enc_W, enc_b2, w1c, b1, w2, b2)
      else:
        m, ef[hh] = _edge_mlp(g[hh], ef[hh], w1c, b1, w2, b2)
      p[hh] = _scatter_add(m, dst3d_h[hh], zrows)
    nb1 = n_b1[i].reshape(1, H)
    nb2 = n_b2[i].reshape(1, H)
    if i < NLAYERS - 1:
      x, x1t, x2t = _node_mlp(x, p[0], p[1], cnts, n_W1[i, :H], n_W1[i, H:],
                              nb1, n_W2[i], nb2, e_W1[i + 1, :H],
                              e_W1[i + 1, H:2 * H])
      for hh in range(2):
        g[hh] = _gather_sum(x1t, x2t, dst_h[hh], src_h[hh])
    else:
      (x,) = _node_mlp(x, p[0], p[1], cnts, n_W1[i, :H], n_W1[i, H:], nb1,
                       n_W2[i], nb2, None, None)
  out = _decode(x, dec_w_pad, dec_b_pad)
  return out[:, :3]
